# baseline (speedup 1.0000x reference)
import functools

import jax
import jax.numpy as jnp
from jax import lax
from jax.experimental import pallas as pl
from jax.experimental.pallas import tpu as pltpu
from jax.experimental.pallas import tpu_sc as plsc

VOCAB = 1000000
DIM = 64
B = 16384
NNEG = 20

NC = 2
NS = 16
NW = NC * NS
BPW = B // NW
CB = 128
NCHUNK = BPW // CB

PACK_S = 2048


def _pack_body(c_ref, t_ref, o_ref):
    ct = c_ref[...].astype(jnp.bfloat16)
    tt = t_ref[...].astype(jnp.bfloat16)
    eye = jnp.eye(DIM, dtype=jnp.bfloat16)
    dn = (((0,), (0,)), ((), ()))
    ctt = jax.lax.dot_general(ct, eye, dn, preferred_element_type=jnp.float32)
    ttt = jax.lax.dot_general(tt, eye, dn, preferred_element_type=jnp.float32)
    o_ref[...] = jnp.concatenate([ctt, ttt], axis=1).astype(jnp.bfloat16)


def _tc_pack(ct, tt):
    return pl.pallas_call(
        _pack_body,
        grid=(pl.cdiv(VOCAB, PACK_S),),
        in_specs=[
            pl.BlockSpec((DIM, PACK_S), lambda i: (0, i)),
            pl.BlockSpec((DIM, PACK_S), lambda i: (0, i)),
        ],
        out_specs=pl.BlockSpec((PACK_S, 2 * DIM), lambda i: (i, 0)),
        out_shape=jax.ShapeDtypeStruct((VOCAB, 2 * DIM), jnp.bfloat16),
    )(ct, tt)


def _sc_body(ptab, t2, cp2, cn3, p_out, n_out,
             idx_t, idx_cp, idx_cn, tbuf, cpbuf, accbuf, pbuf, nbuf,
             sem_idx, sem_g, sem_out):
    ci = lax.axis_index("c")
    si = lax.axis_index("s")
    wid = si * NC + ci
    row4 = wid * NCHUNK

    hts = [
        pltpu.async_copy(t2.at[pl.ds(row4, NCHUNK), :], idx_t, sem_idx),
        pltpu.async_copy(cp2.at[pl.ds(row4, NCHUNK), :], idx_cp, sem_idx),
    ]
    hts += [
        pltpu.async_copy(cn3.at[n, pl.ds(row4, NCHUNK), :], idx_cn.at[n], sem_idx)
        for n in range(NNEG)
    ]
    for h in hts:
        h.wait()

    zero16 = jnp.zeros((16,), jnp.float32)
    zero32 = jnp.zeros((32,), jnp.bfloat16)

    def _unpack2(ref, i, lane):
        pair = ref[i, pl.ds(lane, 32)]
        return plsc.unpack(pair, format=plsc.PackFormat.INTERLEAVED)

    for c in range(NCHUNK):
        def _zbody(i, carry):
            for k in range(4):
                accbuf[i, pl.ds(k * 32, 32)] = zero32
            return carry
        lax.fori_loop(0, CB, _zbody, 0)

        hs = [
            pltpu.async_copy(ptab.at[idx_t.at[c]], tbuf, sem_g),
            pltpu.async_copy(ptab.at[idx_cp.at[c]], cpbuf, sem_g),
        ]
        hs += [
            pltpu.async_copy(ptab.at[idx_cn.at[n, c]], accbuf, sem_g, add=True)
            for n in range(NNEG)
        ]
        for h in hs:
            h.wait()

        def _cbody(i, carry):
            pv = zero16
            nv = zero16
            for k in range(2):
                t0, t1 = _unpack2(tbuf, i, DIM + k * 32)
                c0, c1 = _unpack2(cpbuf, i, k * 32)
                a0, a1 = _unpack2(accbuf, i, k * 32)
                pv = pv + t0 * c0 + t1 * c1
                nv = nv + t0 * a0 + t1 * a1
            pbuf[c * CB + i, :] = pv
            nbuf[c * CB + i, :] = nv
            return carry
        lax.fori_loop(0, CB, _cbody, 0)

    ho = [
        pltpu.async_copy(pbuf, p_out.at[pl.ds(wid * BPW, BPW), :], sem_out),
        pltpu.async_copy(nbuf, n_out.at[pl.ds(wid * BPW, BPW), :], sem_out),
    ]
    for h in ho:
        h.wait()


_sc_scores = functools.partial(
    pl.kernel,
    out_type=(
        jax.ShapeDtypeStruct((B, 16), jnp.float32),
        jax.ShapeDtypeStruct((B, 16), jnp.float32),
    ),
    mesh=plsc.VectorSubcoreMesh(core_axis_name="c", subcore_axis_name="s"),
    compiler_params=pltpu.CompilerParams(
        use_tc_tiling_on_sc=False, needs_layout_passes=False),
    scratch_types=[
        pltpu.VMEM((NCHUNK, CB), jnp.int32),
        pltpu.VMEM((NCHUNK, CB), jnp.int32),
        pltpu.VMEM((NNEG, NCHUNK, CB), jnp.int32),
        pltpu.VMEM((CB, 2 * DIM), jnp.bfloat16),
        pltpu.VMEM((CB, 2 * DIM), jnp.bfloat16),
        pltpu.VMEM((CB, 2 * DIM), jnp.bfloat16),
        pltpu.VMEM((BPW, 16), jnp.float32),
        pltpu.VMEM((BPW, 16), jnp.float32),
        pltpu.SemaphoreType.DMA,
        pltpu.SemaphoreType.DMA,
        pltpu.SemaphoreType.DMA,
    ],
)(_sc_body)


def _log_sigmoid(x):
    return jnp.minimum(x, 0.0) - jnp.log1p(jnp.exp(-jnp.abs(x)))


def _tc_body(p_ref, n_ref, o_ref):
    sp = jnp.sum(p_ref[...], axis=1, keepdims=True)
    sn = jnp.sum(n_ref[...], axis=1, keepdims=True)
    lp = _log_sigmoid(-sp)
    ln = _log_sigmoid(sn)
    o_ref[0, 0] = jnp.sum(lp + ln) / B


def _tc_finish(p, n):
    return pl.pallas_call(
        _tc_body,
        out_shape=jax.ShapeDtypeStruct((1, 1), jnp.float32),
        out_specs=pl.BlockSpec(memory_space=pltpu.SMEM),
    )(p, n)


def kernel(t_vocab_embs, c_vocab_embs, t, cp, cn):
    ptab = _tc_pack(c_vocab_embs.T, t_vocab_embs.T)
    t2 = t.astype(jnp.int32).reshape(B // CB, CB)
    cp2 = cp.astype(jnp.int32).reshape(B // CB, CB)
    cn3 = cn.astype(jnp.int32).T.reshape(NNEG, B // CB, CB)
    p, n = _sc_scores(ptab, t2, cp2, cn3)
    return _tc_finish(p, n)[0, 0]


# speedup vs baseline: 2.6091x; 2.6091x over previous
import functools

import jax
import jax.numpy as jnp
from jax import lax
from jax.experimental import pallas as pl
from jax.experimental.pallas import tpu as pltpu
from jax.experimental.pallas import tpu_sc as plsc

VOCAB = 1000000
DIM = 64
B = 16384
NNEG = 20

NC = 2
NS = 16
NW = NC * NS
BPW = B // NW
CB = 128
NCHUNK = BPW // CB
NSEG = DIM // 16

PACK_S = 8192


def _pack_body(c_ref, t_ref, o_ref):
    ct = c_ref[...]
    tt = t_ref[...]
    o_ref[...] = jnp.concatenate([ct.T, tt.T], axis=1)


def _tc_pack(ct, tt):
    return pl.pallas_call(
        _pack_body,
        grid=(pl.cdiv(VOCAB, PACK_S),),
        in_specs=[
            pl.BlockSpec((DIM, PACK_S), lambda i: (0, i)),
            pl.BlockSpec((DIM, PACK_S), lambda i: (0, i)),
        ],
        out_specs=pl.BlockSpec((PACK_S, 2 * DIM), lambda i: (i, 0)),
        out_shape=jax.ShapeDtypeStruct((VOCAB, 2 * DIM), jnp.float32),
    )(ct, tt)


def _sc_body(ptab, t2, cp2, cn3, p_out, n_out,
             idx_t, idx_cp, idx_cn, tbuf, cpbuf, accbuf, pbuf, nbuf,
             sem_idx, sem_g, sem_out):
    ci = lax.axis_index("c")
    si = lax.axis_index("s")
    wid = si * NC + ci
    row4 = wid * NCHUNK

    hts = [
        pltpu.async_copy(t2.at[pl.ds(row4, NCHUNK), :], idx_t, sem_idx),
        pltpu.async_copy(cp2.at[pl.ds(row4, NCHUNK), :], idx_cp, sem_idx),
    ]
    hts += [
        pltpu.async_copy(cn3.at[n, pl.ds(row4, NCHUNK), :], idx_cn.at[n], sem_idx)
        for n in range(NNEG)
    ]
    for h in hts:
        h.wait()

    zero = jnp.zeros((16,), jnp.float32)

    for c in range(NCHUNK):
        def _zbody(i, carry):
            for k in range(2 * NSEG):
                accbuf[i, pl.ds(k * 16, 16)] = zero
            return carry
        lax.fori_loop(0, CB, _zbody, 0)

        hs = [
            pltpu.async_copy(ptab.at[idx_t.at[c]], tbuf, sem_g),
            pltpu.async_copy(ptab.at[idx_cp.at[c]], cpbuf, sem_g),
        ]
        hs += [
            pltpu.async_copy(ptab.at[idx_cn.at[n, c]], accbuf, sem_g, add=True)
            for n in range(NNEG)
        ]
        for h in hs:
            h.wait()

        def _cbody(i, carry):
            pv = zero
            nv = zero
            for k in range(NSEG):
                tk = tbuf[i, pl.ds(DIM + k * 16, 16)]
                pv = pv + tk * cpbuf[i, pl.ds(k * 16, 16)]
                nv = nv + tk * accbuf[i, pl.ds(k * 16, 16)]
            pbuf[c * CB + i, :] = pv
            nbuf[c * CB + i, :] = nv
            return carry
        lax.fori_loop(0, CB, _cbody, 0)

    ho = [
        pltpu.async_copy(pbuf, p_out.at[pl.ds(wid * BPW, BPW), :], sem_out),
        pltpu.async_copy(nbuf, n_out.at[pl.ds(wid * BPW, BPW), :], sem_out),
    ]
    for h in ho:
        h.wait()


_sc_scores = functools.partial(
    pl.kernel,
    out_type=(
        jax.ShapeDtypeStruct((B, 16), jnp.float32),
        jax.ShapeDtypeStruct((B, 16), jnp.float32),
    ),
    mesh=plsc.VectorSubcoreMesh(core_axis_name="c", subcore_axis_name="s"),
    compiler_params=pltpu.CompilerParams(use_tc_tiling_on_sc=False),
    scratch_types=[
        pltpu.VMEM((NCHUNK, CB), jnp.int32),
        pltpu.VMEM((NCHUNK, CB), jnp.int32),
        pltpu.VMEM((NNEG, NCHUNK, CB), jnp.int32),
        pltpu.VMEM((CB, 2 * DIM), jnp.float32),
        pltpu.VMEM((CB, 2 * DIM), jnp.float32),
        pltpu.VMEM((CB, 2 * DIM), jnp.float32),
        pltpu.VMEM((BPW, 16), jnp.float32),
        pltpu.VMEM((BPW, 16), jnp.float32),
        pltpu.SemaphoreType.DMA,
        pltpu.SemaphoreType.DMA,
        pltpu.SemaphoreType.DMA,
    ],
)(_sc_body)


def _log_sigmoid(x):
    return jnp.minimum(x, 0.0) - jnp.log1p(jnp.exp(-jnp.abs(x)))


def _tc_body(p_ref, n_ref, o_ref):
    sp = jnp.sum(p_ref[...], axis=1, keepdims=True)
    sn = jnp.sum(n_ref[...], axis=1, keepdims=True)
    lp = _log_sigmoid(-sp)
    ln = _log_sigmoid(sn)
    o_ref[0, 0] = jnp.sum(lp + ln) / B


def _tc_finish(p, n):
    return pl.pallas_call(
        _tc_body,
        out_shape=jax.ShapeDtypeStruct((1, 1), jnp.float32),
        out_specs=pl.BlockSpec(memory_space=pltpu.SMEM),
    )(p, n)


def kernel(t_vocab_embs, c_vocab_embs, t, cp, cn):
    ptab = _tc_pack(c_vocab_embs.T, t_vocab_embs.T)
    t2 = t.astype(jnp.int32).reshape(B // CB, CB)
    cp2 = cp.astype(jnp.int32).reshape(B // CB, CB)
    cn3 = cn.astype(jnp.int32).T.reshape(NNEG, B // CB, CB)
    p, n = _sc_scores(ptab, t2, cp2, cn3)
    return _tc_finish(p, n)[0, 0]


# speedup vs baseline: 2.7590x; 1.0575x over previous
import functools

import jax
import jax.numpy as jnp
from jax import lax
from jax.experimental import pallas as pl
from jax.experimental.pallas import tpu as pltpu
from jax.experimental.pallas import tpu_sc as plsc

VOCAB = 1000000
DIM = 64
B = 16384
NNEG = 20

NC = 2
NS = 16
NW = NC * NS
BPW = B // NW
CB = 128
NCHUNK = BPW // CB
NSEG = DIM // 16

PACK_S = 16384


def _pack_body(c_ref, t_ref, o_ref):
    ct = c_ref[...]
    tt = t_ref[...]
    o_ref[...] = jnp.concatenate([ct.T, tt.T], axis=1)


def _tc_pack(ct, tt):
    return pl.pallas_call(
        _pack_body,
        grid=(pl.cdiv(VOCAB, PACK_S),),
        in_specs=[
            pl.BlockSpec((DIM, PACK_S), lambda i: (0, i)),
            pl.BlockSpec((DIM, PACK_S), lambda i: (0, i)),
        ],
        out_specs=pl.BlockSpec((PACK_S, 2 * DIM), lambda i: (i, 0)),
        out_shape=jax.ShapeDtypeStruct((VOCAB, 2 * DIM), jnp.float32),
    )(ct, tt)


def _sc_body(ptab, t2, cp2, cn3, p_out, n_out,
             idx_t, idx_cp, idx_cn, tbuf, cpbuf, accbuf, pbuf, nbuf,
             sem_idx, sem_g, sem_out):
    ci = lax.axis_index("c")
    si = lax.axis_index("s")
    wid = si * NC + ci
    row4 = wid * NCHUNK

    hts = [
        pltpu.async_copy(t2.at[pl.ds(row4, NCHUNK), :], idx_t, sem_idx),
        pltpu.async_copy(cp2.at[pl.ds(row4, NCHUNK), :], idx_cp, sem_idx),
    ]
    hts += [
        pltpu.async_copy(cn3.at[n, pl.ds(row4, NCHUNK), :], idx_cn.at[n], sem_idx)
        for n in range(NNEG)
    ]
    for h in hts:
        h.wait()

    zero = jnp.zeros((16,), jnp.float32)

    for c in range(NCHUNK):
        def _zbody(i, carry):
            for k in range(2 * NSEG):
                accbuf[i, pl.ds(k * 16, 16)] = zero
            return carry
        lax.fori_loop(0, CB, _zbody, 0)

        hs = [
            pltpu.async_copy(ptab.at[idx_t.at[c]], tbuf, sem_g),
            pltpu.async_copy(ptab.at[idx_cp.at[c]], cpbuf, sem_g),
        ]
        hs += [
            pltpu.async_copy(ptab.at[idx_cn.at[n, c]], accbuf, sem_g, add=True)
            for n in range(NNEG)
        ]
        for h in hs:
            h.wait()

        def _cbody(i, carry):
            pv = zero
            nv = zero
            for k in range(NSEG):
                tk = tbuf[i, pl.ds(DIM + k * 16, 16)]
                pv = pv + tk * cpbuf[i, pl.ds(k * 16, 16)]
                nv = nv + tk * accbuf[i, pl.ds(k * 16, 16)]
            pbuf[c * CB + i, :] = pv
            nbuf[c * CB + i, :] = nv
            return carry
        lax.fori_loop(0, CB, _cbody, 0)

    ho = [
        pltpu.async_copy(pbuf, p_out.at[pl.ds(wid * BPW, BPW), :], sem_out),
        pltpu.async_copy(nbuf, n_out.at[pl.ds(wid * BPW, BPW), :], sem_out),
    ]
    for h in ho:
        h.wait()


_sc_scores = functools.partial(
    pl.kernel,
    out_type=(
        jax.ShapeDtypeStruct((B, 16), jnp.float32),
        jax.ShapeDtypeStruct((B, 16), jnp.float32),
    ),
    mesh=plsc.VectorSubcoreMesh(core_axis_name="c", subcore_axis_name="s"),
    compiler_params=pltpu.CompilerParams(use_tc_tiling_on_sc=False),
    scratch_types=[
        pltpu.VMEM((NCHUNK, CB), jnp.int32),
        pltpu.VMEM((NCHUNK, CB), jnp.int32),
        pltpu.VMEM((NNEG, NCHUNK, CB), jnp.int32),
        pltpu.VMEM((CB, 2 * DIM), jnp.float32),
        pltpu.VMEM((CB, 2 * DIM), jnp.float32),
        pltpu.VMEM((CB, 2 * DIM), jnp.float32),
        pltpu.VMEM((BPW, 16), jnp.float32),
        pltpu.VMEM((BPW, 16), jnp.float32),
        pltpu.SemaphoreType.DMA,
        pltpu.SemaphoreType.DMA,
        pltpu.SemaphoreType.DMA,
    ],
)(_sc_body)


def _log_sigmoid(x):
    return jnp.minimum(x, 0.0) - jnp.log1p(jnp.exp(-jnp.abs(x)))


def _tc_body(p_ref, n_ref, o_ref):
    sp = jnp.sum(p_ref[...], axis=1, keepdims=True)
    sn = jnp.sum(n_ref[...], axis=1, keepdims=True)
    lp = _log_sigmoid(-sp)
    ln = _log_sigmoid(sn)
    o_ref[0, 0] = jnp.sum(lp + ln) / B


def _tc_finish(p, n):
    return pl.pallas_call(
        _tc_body,
        out_shape=jax.ShapeDtypeStruct((1, 1), jnp.float32),
        out_specs=pl.BlockSpec(memory_space=pltpu.SMEM),
    )(p, n)


def kernel(t_vocab_embs, c_vocab_embs, t, cp, cn):
    ptab = _tc_pack(c_vocab_embs.T, t_vocab_embs.T)
    t2 = t.astype(jnp.int32).reshape(B // CB, CB)
    cp2 = cp.astype(jnp.int32).reshape(B // CB, CB)
    cn3 = cn.astype(jnp.int32).T.reshape(NNEG, B // CB, CB)
    p, n = _sc_scores(ptab, t2, cp2, cn3)
    return _tc_finish(p, n)[0, 0]


# speedup vs baseline: 2.7904x; 1.0114x over previous
import functools

import jax
import jax.numpy as jnp
from jax import lax
from jax.experimental import pallas as pl
from jax.experimental.pallas import tpu as pltpu
from jax.experimental.pallas import tpu_sc as plsc

VOCAB = 1000000
DIM = 64
B = 16384
NNEG = 20

NC = 2
NS = 16
NW = NC * NS
BPW = B // NW
CB = 64
NCHUNK = BPW // CB
NSEG = DIM // 16

PACK_S = 16384


def _pack_body(c_ref, t_ref, o_ref):
    ct = c_ref[...]
    tt = t_ref[...]
    o_ref[...] = jnp.concatenate([ct.T, tt.T], axis=1)


def _tc_pack(ct, tt):
    return pl.pallas_call(
        _pack_body,
        grid=(pl.cdiv(VOCAB, PACK_S),),
        in_specs=[
            pl.BlockSpec((DIM, PACK_S), lambda i: (0, i)),
            pl.BlockSpec((DIM, PACK_S), lambda i: (0, i)),
        ],
        out_specs=pl.BlockSpec((PACK_S, 2 * DIM), lambda i: (i, 0)),
        out_shape=jax.ShapeDtypeStruct((VOCAB, 2 * DIM), jnp.float32),
    )(ct, tt)


def _sc_body(ptab, t2, cp2, cn3, p_out, n_out,
             idx_t, idx_cp, idx_cn, tbuf, cpbuf, accbuf, pbuf, nbuf,
             sem_idx, sem_g, sem_out):
    ci = lax.axis_index("c")
    si = lax.axis_index("s")
    wid = si * NC + ci
    row4 = wid * NCHUNK

    hts = [
        pltpu.async_copy(t2.at[pl.ds(row4, NCHUNK), :], idx_t, sem_idx),
        pltpu.async_copy(cp2.at[pl.ds(row4, NCHUNK), :], idx_cp, sem_idx),
    ]
    hts += [
        pltpu.async_copy(cn3.at[n, pl.ds(row4, NCHUNK), :], idx_cn.at[n], sem_idx)
        for n in range(NNEG)
    ]
    for h in hts:
        h.wait()

    zero = jnp.zeros((16,), jnp.float32)

    def _fire(c, buf):
        def _zbody(i, carry):
            for k in range(2 * NSEG):
                accbuf[buf, i, pl.ds(k * 16, 16)] = zero
            return carry
        lax.fori_loop(0, CB, _zbody, 0)
        hs = [
            pltpu.async_copy(ptab.at[idx_t.at[c]], tbuf.at[buf], sem_g.at[buf]),
            pltpu.async_copy(ptab.at[idx_cp.at[c]], cpbuf.at[buf], sem_g.at[buf]),
        ]
        hs += [
            pltpu.async_copy(ptab.at[idx_cn.at[n, c]], accbuf.at[buf],
                             sem_g.at[buf], add=True)
            for n in range(NNEG)
        ]
        return hs

    def _compute(c, buf):
        def _cbody(i, carry):
            pv = zero
            nv = zero
            for k in range(NSEG):
                tk = tbuf[buf, i, pl.ds(DIM + k * 16, 16)]
                pv = pv + tk * cpbuf[buf, i, pl.ds(k * 16, 16)]
                nv = nv + tk * accbuf[buf, i, pl.ds(k * 16, 16)]
            pbuf[c * CB + i, :] = pv
            nbuf[c * CB + i, :] = nv
            return carry
        lax.fori_loop(0, CB, _cbody, 0)

    inflight = _fire(0, 0)
    for c in range(1, NCHUNK + 1):
        if c < NCHUNK:
            nxt = _fire(c, c % 2)
        for h in inflight:
            h.wait()
        _compute(c - 1, (c - 1) % 2)
        if c < NCHUNK:
            inflight = nxt

    ho = [
        pltpu.async_copy(pbuf, p_out.at[pl.ds(wid * BPW, BPW), :], sem_out),
        pltpu.async_copy(nbuf, n_out.at[pl.ds(wid * BPW, BPW), :], sem_out),
    ]
    for h in ho:
        h.wait()


_sc_scores = functools.partial(
    pl.kernel,
    out_type=(
        jax.ShapeDtypeStruct((B, 16), jnp.float32),
        jax.ShapeDtypeStruct((B, 16), jnp.float32),
    ),
    mesh=plsc.VectorSubcoreMesh(core_axis_name="c", subcore_axis_name="s"),
    compiler_params=pltpu.CompilerParams(use_tc_tiling_on_sc=False),
    scratch_types=[
        pltpu.VMEM((NCHUNK, CB), jnp.int32),
        pltpu.VMEM((NCHUNK, CB), jnp.int32),
        pltpu.VMEM((NNEG, NCHUNK, CB), jnp.int32),
        pltpu.VMEM((2, CB, 2 * DIM), jnp.float32),
        pltpu.VMEM((2, CB, 2 * DIM), jnp.float32),
        pltpu.VMEM((2, CB, 2 * DIM), jnp.float32),
        pltpu.VMEM((BPW, 16), jnp.float32),
        pltpu.VMEM((BPW, 16), jnp.float32),
        pltpu.SemaphoreType.DMA,
        pltpu.SemaphoreType.DMA((2,)),
        pltpu.SemaphoreType.DMA,
    ],
)(_sc_body)


def _log_sigmoid(x):
    return jnp.minimum(x, 0.0) - jnp.log1p(jnp.exp(-jnp.abs(x)))


def _tc_body(p_ref, n_ref, o_ref):
    sp = jnp.sum(p_ref[...], axis=1, keepdims=True)
    sn = jnp.sum(n_ref[...], axis=1, keepdims=True)
    lp = _log_sigmoid(-sp)
    ln = _log_sigmoid(sn)
    o_ref[0, 0] = jnp.sum(lp + ln) / B


def _tc_finish(p, n):
    return pl.pallas_call(
        _tc_body,
        out_shape=jax.ShapeDtypeStruct((1, 1), jnp.float32),
        out_specs=pl.BlockSpec(memory_space=pltpu.SMEM),
    )(p, n)


def kernel(t_vocab_embs, c_vocab_embs, t, cp, cn):
    ptab = _tc_pack(c_vocab_embs.T, t_vocab_embs.T)
    t2 = t.astype(jnp.int32).reshape(B // CB, CB)
    cp2 = cp.astype(jnp.int32).reshape(B // CB, CB)
    cn3 = cn.astype(jnp.int32).T.reshape(NNEG, B // CB, CB)
    p, n = _sc_scores(ptab, t2, cp2, cn3)
    return _tc_finish(p, n)[0, 0]
